# Initial kernel scaffold; baseline (speedup 1.0000x reference)
import jax
import jax.numpy as jnp
from jax.experimental import pallas as pl


def kernel(batch_confidence_maps, B, batch_warp_maks_list, gw, gb):
    raise NotImplementedError("write your pallas kernel here")


# speedup vs baseline: 40.5306x; 40.5306x over previous
import jax
import jax.numpy as jnp
from jax import lax
from jax.experimental import pallas as pl
from jax.experimental.pallas import tpu as pltpu

_H = 512
_W = 512
_L = 6
_KTAP = 5
_BIS_ITERS = 32


def _row_body(gw_ref, gb_ref, conf_ref, warp_ref, mask_ref, cnt_ref):
    is_ego = (pl.program_id(0) % _L) == 0
    K = jnp.float32(_H * _W // 2)

    @pl.when(jnp.logical_not(is_ego))
    def _():
        c0 = conf_ref[0, 0, 0]
        c1 = conf_ref[0, 0, 1]
        s = jnp.maximum(jax.nn.sigmoid(c0), jax.nn.sigmoid(c1))
        s = s * warp_ref[0, 0]
        s = s.astype(jnp.bfloat16).astype(jnp.float32)
        zc = jnp.zeros((_H, 2), jnp.float32)
        sp = jnp.concatenate([zc, s, zc], axis=1)
        zr = jnp.zeros((2, _W + 4), jnp.float32)
        sp = jnp.concatenate([zr, sp, zr], axis=0)
        acc = jnp.full((_H, _W), gb_ref[0], jnp.float32)
        for dy in range(_KTAP):
            for dx in range(_KTAP):
                acc = acc + gw_ref[dy * _KTAP + dx] * sp[dy:dy + _H, dx:dx + _W]

        def bis(_, c):
            lo, hi = c
            mid = (lo + hi) * jnp.float32(0.5)
            cnt = jnp.sum((acc > mid).astype(jnp.float32))
            ok = cnt >= K
            return (jnp.where(ok, mid, lo), jnp.where(ok, hi, mid))

        lo, _hi = lax.fori_loop(0, _BIS_ITERS, bis,
                                (jnp.float32(-1.0), jnp.float32(2.0)))
        m = (acc > lo).astype(jnp.float32)
        mask_ref[0] = m
        cnt_ref[0] = jnp.full((8, 128), jnp.sum(m), jnp.float32)

    @pl.when(is_ego)
    def _():
        mask_ref[0] = jnp.ones((_H, _W), jnp.float32)
        cnt_ref[0] = jnp.full((8, 128), K, jnp.float32)


def _rtne_bf16_f32(x):
    u = lax.bitcast_convert_type(x, jnp.uint32)
    u = (u + jnp.uint32(0x7FFF) + ((u >> 16) & jnp.uint32(1))) & jnp.uint32(0xFFFF0000)
    return lax.bitcast_convert_type(u, jnp.float32)


def kernel(batch_confidence_maps, B, batch_warp_maks_list, gw, gb):
    Bs, L, C, H, W = batch_confidence_maps.shape
    gwf = _rtne_bf16_f32(gw.reshape(_KTAP * _KTAP))

    masks, counts = pl.pallas_call(
        _row_body,
        grid=(Bs * L,),
        in_specs=[
            pl.BlockSpec(memory_space=pltpu.SMEM),
            pl.BlockSpec(memory_space=pltpu.SMEM),
            pl.BlockSpec((1, 1, C, H, W), lambda i: (i // L, i % L, 0, 0, 0)),
            pl.BlockSpec((1, 1, H, W), lambda i: (i // L, 0, 0, 0)),
        ],
        out_specs=[
            pl.BlockSpec((1, H, W), lambda i: (i, 0, 0)),
            pl.BlockSpec((1, 8, 128), lambda i: (i, 0, 0)),
        ],
        out_shape=[
            jax.ShapeDtypeStruct((Bs * L, H, W), jnp.float32),
            jax.ShapeDtypeStruct((Bs * L, 8, 128), jnp.float32),
        ],
        compiler_params=pltpu.CompilerParams(
            dimension_semantics=("arbitrary",)),
    )(gwf, gb, batch_confidence_maps, batch_warp_maks_list)

    comm_masks = masks.reshape(Bs * L, 1, H, W)
    cnt = counts[:, 0, 0].reshape(Bs, L)[:, 1:].sum()
    rate = cnt / jnp.float32(Bs * (L - 1) * H * W)
    return comm_masks, rate
